# baseline (speedup 1.0000x reference)
import jax
import jax.numpy as jnp
from jax.experimental import pallas as pl
from jax.experimental.pallas import tpu as pltpu

USER_ROWS = 1_000_000 * 16 // 128
ITEM_ROWS = 100_000 * 16 // 128
GRID = 25
U_BLK = USER_ROWS // GRID
I_BLK = ITEM_ROWS // GRID


def _copy_body(u_in, i_in, u_out, i_out):
    u_out[...] = u_in[...]
    i_out[...] = i_in[...]


def kernel(user_emb, item_emb):
    u = user_emb.reshape(USER_ROWS, 128)
    it = item_emb.reshape(GRID, I_BLK, 128)
    u_o, i_o = pl.pallas_call(
        _copy_body,
        grid=(GRID,),
        in_specs=[
            pl.BlockSpec((U_BLK, 128), lambda i: (i, 0)),
            pl.BlockSpec((1, I_BLK, 128), lambda i: (i, 0, 0)),
        ],
        out_specs=[
            pl.BlockSpec((U_BLK, 128), lambda i: (i, 0)),
            pl.BlockSpec((1, I_BLK, 128), lambda i: (i, 0, 0)),
        ],
        out_shape=[
            jax.ShapeDtypeStruct((USER_ROWS, 128), jnp.float32),
            jax.ShapeDtypeStruct((GRID, I_BLK, 128), jnp.float32),
        ],
    )(u, it)
    return u_o.reshape(user_emb.shape), i_o.reshape(item_emb.shape)
